# Initial kernel scaffold; baseline (speedup 1.0000x reference)
import jax
import jax.numpy as jnp
from jax.experimental import pallas as pl


def kernel(feat, edge_index, W_self0, W_neigh0, b0, ln_gamma, ln_beta, W_self1, W_neigh1, b1):
    raise NotImplementedError("write your pallas kernel here")


# speedup vs baseline: 6.3803x; 6.3803x over previous
import functools

import jax
import jax.numpy as jnp
from jax import lax
from jax.experimental import pallas as pl
from jax.experimental.pallas import tpu as pltpu
from jax.experimental.pallas import tpu_sc as plsc

_N = 10000
_E = 320000
_D = 128
_NC = 2
_NS = 16
_NW = _NC * _NS
_EPW = _E // _NW
_CHUNK = 80
_NCHUNK = _EPW // _CHUNK
_WB = 200
_NWB = _N // _WB


def _seg_body(with_deg, *refs):
    if with_deg:
        (val_hbm, src_hbm, dst_hbm, part_hbm, deg0_hbm, deg1_hbm,
         acc_sh, deg_sh, sidx, didx, rows, stage, dstage, ones, sem) = refs
    else:
        (val_hbm, src_hbm, dst_hbm, part_hbm,
         acc_sh, sidx, didx, rows, stage, sem) = refs
    c = lax.axis_index("c")
    s = lax.axis_index("s")
    wid = s * _NC + c

    zv = jnp.zeros((16,), jnp.float32)

    def _zrow(r, _):
        def _zcol(q, _2):
            stage[r, pl.ds(q * 16, 16)] = zv
            return 0
        lax.fori_loop(0, _D // 16, _zcol, 0)
        return 0
    lax.fori_loop(0, _WB, _zrow, 0)

    for k in range(4):
        idx = s + _NS * k
        @pl.when(idx < _NWB)
        def _():
            pltpu.sync_copy(stage, acc_sh.at[pl.ds(idx * _WB, _WB)])

    if with_deg:
        def _zdeg(i, _):
            dstage[pl.ds(i * 16, 16)] = zv
            return 0
        lax.fori_loop(0, 63, _zdeg, 0)
        def _fones(i, _):
            ones[pl.ds(i * 16, 16)] = zv + 1.0
            return 0
        lax.fori_loop(0, _CHUNK // 16, _fones, 0)

        @pl.when(s < 10)
        def _():
            pltpu.sync_copy(dstage.at[pl.ds(0, 1000)],
                            deg_sh.at[pl.ds(s * 1000, 1000)])

    plsc.subcore_barrier()

    base = wid * _EPW

    def _step(j, _):
        off = base + j * _CHUNK
        pltpu.sync_copy(src_hbm.at[pl.ds(off, _CHUNK)], sidx)
        pltpu.sync_copy(dst_hbm.at[pl.ds(off, _CHUNK)], didx)
        pltpu.async_copy(val_hbm.at[sidx], rows, sem).wait()
        pltpu.sync_copy(rows, acc_sh.at[didx], add=True)
        if with_deg:
            pltpu.sync_copy(ones, deg_sh.at[didx], add=True)
        return 0
    lax.fori_loop(0, _NCHUNK, _step, 0)

    plsc.subcore_barrier()

    for k in range(4):
        idx = s + _NS * k
        @pl.when(idx < _NWB)
        def _():
            pltpu.sync_copy(acc_sh.at[pl.ds(idx * _WB, _WB)], stage)
            pltpu.sync_copy(stage, part_hbm.at[c, pl.ds(idx * _WB, _WB)])

    if with_deg:
        @pl.when(s < 10)
        def _():
            pltpu.sync_copy(deg_sh.at[pl.ds(s * 1000, 1000)],
                            dstage.at[pl.ds(0, 1000)])

            @pl.when(c == 0)
            def _():
                pltpu.sync_copy(dstage.at[pl.ds(0, 1000)],
                                deg0_hbm.at[pl.ds(s * 1000, 1000)])

            @pl.when(c == 1)
            def _():
                pltpu.sync_copy(dstage.at[pl.ds(0, 1000)],
                                deg1_hbm.at[pl.ds(s * 1000, 1000)])


def _make_seg_kernel(with_deg):
    out_type = [jax.ShapeDtypeStruct((_NC, _N, _D), jnp.float32)]
    scratch = [
        pltpu.VMEM_SHARED((_N, _D), jnp.float32),
    ]
    if with_deg:
        out_type.append(jax.ShapeDtypeStruct((_N,), jnp.float32))
        out_type.append(jax.ShapeDtypeStruct((_N,), jnp.float32))
        scratch.append(pltpu.VMEM_SHARED((_N,), jnp.float32))
    scratch += [
        pltpu.VMEM((_CHUNK,), jnp.int32),
        pltpu.VMEM((_CHUNK,), jnp.int32),
        pltpu.VMEM((_CHUNK, _D), jnp.float32),
        pltpu.VMEM((_WB, _D), jnp.float32),
    ]
    if with_deg:
        scratch += [
            pltpu.VMEM((1008,), jnp.float32),
            pltpu.VMEM((_CHUNK,), jnp.float32),
        ]
    scratch.append(pltpu.SemaphoreType.DMA)
    mesh = plsc.VectorSubcoreMesh(core_axis_name="c", subcore_axis_name="s",
                                  num_cores=_NC, num_subcores=_NS)
    return pl.kernel(functools.partial(_seg_body, with_deg),
                     out_type=out_type, mesh=mesh, scratch_types=scratch)


_seg_with_deg = _make_seg_kernel(True)
_seg_no_deg = _make_seg_kernel(False)


def _dense1_body(feat_ref, p_ref, degt_ref, ws0_ref, wn0_ref, b0_ref,
                 g_ref, be_ref, ws1_ref, wn1_ref, hs_ref, hw_ref):
    deg = degt_ref[...]
    inv = 1.0 / jnp.maximum(deg[:, 0:1] + deg[:, 1:2], 1.0)
    agg = (p_ref[0] + p_ref[1]) * inv
    z = (jnp.dot(feat_ref[...], ws0_ref[...],
                 preferred_element_type=jnp.float32)
         + jnp.dot(agg, wn0_ref[...], preferred_element_type=jnp.float32)
         + b0_ref[...])
    mu = jnp.mean(z, axis=-1, keepdims=True)
    var = jnp.mean((z - mu) ** 2, axis=-1, keepdims=True)
    h = (z - mu) * lax.rsqrt(var + 1e-5) * g_ref[...] + be_ref[...]
    h = jnp.maximum(h, 0.0)
    hs_ref[...] = jnp.dot(h, ws1_ref[...], preferred_element_type=jnp.float32)
    hw_ref[...] = jnp.dot(h, wn1_ref[...], preferred_element_type=jnp.float32)


def _dense2_body(hs_ref, q_ref, degt_ref, b1_ref, out_ref):
    deg = degt_ref[...]
    inv = 1.0 / jnp.maximum(deg[:, 0:1] + deg[:, 1:2], 1.0)
    out_ref[...] = hs_ref[...] + (q_ref[0] + q_ref[1]) * inv + b1_ref[...]


_BLK = 1000
_GRID = _N // _BLK


def _dense1(feat, p, degt, ws0, wn0, b0, g, be, ws1, wn1):
    return pl.pallas_call(
        _dense1_body,
        grid=(_GRID,),
        in_specs=[
            pl.BlockSpec((_BLK, 128), lambda i: (i, 0)),
            pl.BlockSpec((_NC, _BLK, 128), lambda i: (0, i, 0)),
            pl.BlockSpec((_BLK, _NC), lambda i: (i, 0)),
            pl.BlockSpec((128, 256), lambda i: (0, 0)),
            pl.BlockSpec((128, 256), lambda i: (0, 0)),
            pl.BlockSpec((1, 256), lambda i: (0, 0)),
            pl.BlockSpec((1, 256), lambda i: (0, 0)),
            pl.BlockSpec((1, 256), lambda i: (0, 0)),
            pl.BlockSpec((256, 128), lambda i: (0, 0)),
            pl.BlockSpec((256, 128), lambda i: (0, 0)),
        ],
        out_specs=[
            pl.BlockSpec((_BLK, 128), lambda i: (i, 0)),
            pl.BlockSpec((_BLK, 128), lambda i: (i, 0)),
        ],
        out_shape=[
            jax.ShapeDtypeStruct((_N, 128), jnp.float32),
            jax.ShapeDtypeStruct((_N, 128), jnp.float32),
        ],
    )(feat, p, degt, ws0, wn0, b0, g, be, ws1, wn1)


def _dense2(hs, q, degt, b1):
    return pl.pallas_call(
        _dense2_body,
        grid=(_GRID,),
        in_specs=[
            pl.BlockSpec((_BLK, 128), lambda i: (i, 0)),
            pl.BlockSpec((_NC, _BLK, 128), lambda i: (0, i, 0)),
            pl.BlockSpec((_BLK, _NC), lambda i: (i, 0)),
            pl.BlockSpec((1, 128), lambda i: (0, 0)),
        ],
        out_specs=pl.BlockSpec((_BLK, 128), lambda i: (i, 0)),
        out_shape=jax.ShapeDtypeStruct((_N, 128), jnp.float32),
    )(hs, q, degt, b1)


def kernel(feat, edge_index, W_self0, W_neigh0, b0, ln_gamma, ln_beta,
           W_self1, W_neigh1, b1):
    src = edge_index[0]
    dst = edge_index[1]
    p1, deg0, deg1 = _seg_with_deg(feat, src, dst)
    degt = jnp.concatenate([deg0[:, None], deg1[:, None]], axis=1)
    hs, hw = _dense1(feat, p1, degt, W_self0, W_neigh0,
                     b0.reshape(1, -1), ln_gamma.reshape(1, -1),
                     ln_beta.reshape(1, -1), W_self1, W_neigh1)
    (q,) = _seg_no_deg(hw, src, dst)
    return _dense2(hs, q, degt, b1.reshape(1, -1))


# speedup vs baseline: 11.6843x; 1.8313x over previous
import functools

import jax
import jax.numpy as jnp
from jax import lax
from jax.experimental import pallas as pl
from jax.experimental.pallas import tpu as pltpu
from jax.experimental.pallas import tpu_sc as plsc

_N = 10000
_E = 320000
_D = 128
_NC = 2
_NS = 16
_NW = _NC * _NS
_EPW = _E // _NW
_CHUNK = 80
_NCHUNK = _EPW // _CHUNK
_WB = 80
_NWB = _N // _WB


def _seg_body(with_deg, *refs):
    if with_deg:
        (val_hbm, src_hbm, dst_hbm, part_hbm, deg0_hbm, deg1_hbm,
         acc_sh, deg_sh, sidx, didx, rows, dstage, ones,
         gsem, ssem, dsem) = refs
    else:
        (val_hbm, src_hbm, dst_hbm, part_hbm,
         acc_sh, sidx, didx, rows, gsem, ssem) = refs
    c = lax.axis_index("c")
    s = lax.axis_index("s")
    wid = s * _NC + c

    zv = jnp.zeros((16,), jnp.float32)

    pltpu.sync_copy(src_hbm.at[wid], sidx)
    pltpu.sync_copy(dst_hbm.at[wid], didx)

    def _zrow(r, _):
        def _zcol(q, _2):
            rows[0, r, pl.ds(q * 16, 16)] = zv
            return 0
        lax.fori_loop(0, _D // 16, _zcol, 0)
        return 0
    lax.fori_loop(0, _CHUNK, _zrow, 0)

    for k in range(8):
        idx = s + _NS * k
        @pl.when(idx < _NWB)
        def _():
            pltpu.sync_copy(rows.at[0], acc_sh.at[pl.ds(idx * _WB, _WB)])

    if with_deg:
        def _zdeg(i, _):
            dstage[pl.ds(i * 16, 16)] = zv
            return 0
        lax.fori_loop(0, 63, _zdeg, 0)
        def _fones(i, _):
            ones[pl.ds(i * 16, 16)] = zv + 1.0
            return 0
        lax.fori_loop(0, _CHUNK // 16, _fones, 0)

        @pl.when(s < 10)
        def _():
            pltpu.sync_copy(dstage.at[pl.ds(0, 1000)],
                            deg_sh.at[pl.ds(s * 1000, 1000)])

    plsc.subcore_barrier()

    def _g_desc(j, b):
        return pltpu.make_async_copy(
            val_hbm.at[sidx.at[pl.ds(j * _CHUNK, _CHUNK)]], rows.at[b],
            gsem.at[b])

    def _s_desc(j, b):
        return pltpu.make_async_copy(rows.at[b], acc_sh.at[didx.at[j]],
                                     ssem.at[b])

    def _d_desc(j, b):
        return pltpu.make_async_copy(ones, deg_sh.at[didx.at[j]],
                                     dsem.at[b])

    def _slot(j, b, drain, fire):
        _g_desc(j, b).wait()
        _s_desc(j, b).start(add=True)
        if with_deg:
            _d_desc(j, b).start(add=True)
        if drain:
            _s_desc(j - 1, 1 - b).wait()
            if with_deg:
                _d_desc(j - 1, 1 - b).wait()
        if fire:
            _g_desc(j + 1, 1 - b).start()

    _g_desc(0, 0).start()
    _slot(0, 0, False, True)

    def _mid(t, _):
        _slot(2 * t + 1, 1, True, True)
        _slot(2 * t + 2, 0, True, True)
        return 0
    lax.fori_loop(0, (_NCHUNK - 3) // 2, _mid, 0)

    _slot(_NCHUNK - 2, 1, True, True)
    _slot(_NCHUNK - 1, 0, True, False)
    _s_desc(_NCHUNK - 1, 0).wait()
    if with_deg:
        _d_desc(_NCHUNK - 1, 0).wait()

    plsc.subcore_barrier()

    for k in range(8):
        idx = s + _NS * k
        @pl.when(idx < _NWB)
        def _():
            pltpu.sync_copy(acc_sh.at[pl.ds(idx * _WB, _WB)], rows.at[0])
            pltpu.sync_copy(rows.at[0], part_hbm.at[c, pl.ds(idx * _WB, _WB)])

    if with_deg:
        @pl.when(s < 10)
        def _():
            pltpu.sync_copy(deg_sh.at[pl.ds(s * 1000, 1000)],
                            dstage.at[pl.ds(0, 1000)])

            @pl.when(c == 0)
            def _():
                pltpu.sync_copy(dstage.at[pl.ds(0, 1000)],
                                deg0_hbm.at[pl.ds(s * 1000, 1000)])

            @pl.when(c == 1)
            def _():
                pltpu.sync_copy(dstage.at[pl.ds(0, 1000)],
                                deg1_hbm.at[pl.ds(s * 1000, 1000)])


def _make_seg_kernel(with_deg):
    out_type = [jax.ShapeDtypeStruct((_NC, _N, _D), jnp.float32)]
    scratch = [
        pltpu.VMEM_SHARED((_N, _D), jnp.float32),
    ]
    if with_deg:
        out_type.append(jax.ShapeDtypeStruct((_N,), jnp.float32))
        out_type.append(jax.ShapeDtypeStruct((_N,), jnp.float32))
        scratch.append(pltpu.VMEM_SHARED((_N,), jnp.float32))
    scratch += [
        pltpu.VMEM((_EPW,), jnp.int32),
        pltpu.VMEM((_NCHUNK, _CHUNK), jnp.int32),
        pltpu.VMEM((2, _CHUNK, _D), jnp.float32),
    ]
    if with_deg:
        scratch += [
            pltpu.VMEM((1008,), jnp.float32),
            pltpu.VMEM((_CHUNK,), jnp.float32),
        ]
    scratch.append(pltpu.SemaphoreType.DMA((2,)))
    scratch.append(pltpu.SemaphoreType.DMA((2,)))
    if with_deg:
        scratch.append(pltpu.SemaphoreType.DMA((2,)))
    mesh = plsc.VectorSubcoreMesh(core_axis_name="c", subcore_axis_name="s",
                                  num_cores=_NC, num_subcores=_NS)
    return pl.kernel(functools.partial(_seg_body, with_deg),
                     out_type=out_type, mesh=mesh, scratch_types=scratch)


_seg_with_deg = _make_seg_kernel(True)
_seg_no_deg = _make_seg_kernel(False)


def _dense1_body(feat_ref, p_ref, degt_ref, ws0_ref, wn0_ref, b0_ref,
                 g_ref, be_ref, ws1_ref, wn1_ref, hs_ref, hw_ref):
    deg = degt_ref[...]
    inv = 1.0 / jnp.maximum(deg[:, 0:1] + deg[:, 1:2], 1.0)
    agg = (p_ref[0] + p_ref[1]) * inv
    z = (jnp.dot(feat_ref[...], ws0_ref[...],
                 preferred_element_type=jnp.float32)
         + jnp.dot(agg, wn0_ref[...], preferred_element_type=jnp.float32)
         + b0_ref[...])
    mu = jnp.mean(z, axis=-1, keepdims=True)
    var = jnp.mean((z - mu) ** 2, axis=-1, keepdims=True)
    h = (z - mu) * lax.rsqrt(var + 1e-5) * g_ref[...] + be_ref[...]
    h = jnp.maximum(h, 0.0)
    hs_ref[...] = jnp.dot(h, ws1_ref[...], preferred_element_type=jnp.float32)
    hw_ref[...] = jnp.dot(h, wn1_ref[...], preferred_element_type=jnp.float32)


def _dense2_body(hs_ref, q_ref, degt_ref, b1_ref, out_ref):
    deg = degt_ref[...]
    inv = 1.0 / jnp.maximum(deg[:, 0:1] + deg[:, 1:2], 1.0)
    out_ref[...] = hs_ref[...] + (q_ref[0] + q_ref[1]) * inv + b1_ref[...]


_BLK = 1000
_GRID = _N // _BLK


def _dense1(feat, p, degt, ws0, wn0, b0, g, be, ws1, wn1):
    return pl.pallas_call(
        _dense1_body,
        grid=(_GRID,),
        in_specs=[
            pl.BlockSpec((_BLK, 128), lambda i: (i, 0)),
            pl.BlockSpec((_NC, _BLK, 128), lambda i: (0, i, 0)),
            pl.BlockSpec((_BLK, _NC), lambda i: (i, 0)),
            pl.BlockSpec((128, 256), lambda i: (0, 0)),
            pl.BlockSpec((128, 256), lambda i: (0, 0)),
            pl.BlockSpec((1, 256), lambda i: (0, 0)),
            pl.BlockSpec((1, 256), lambda i: (0, 0)),
            pl.BlockSpec((1, 256), lambda i: (0, 0)),
            pl.BlockSpec((256, 128), lambda i: (0, 0)),
            pl.BlockSpec((256, 128), lambda i: (0, 0)),
        ],
        out_specs=[
            pl.BlockSpec((_BLK, 128), lambda i: (i, 0)),
            pl.BlockSpec((_BLK, 128), lambda i: (i, 0)),
        ],
        out_shape=[
            jax.ShapeDtypeStruct((_N, 128), jnp.float32),
            jax.ShapeDtypeStruct((_N, 128), jnp.float32),
        ],
    )(feat, p, degt, ws0, wn0, b0, g, be, ws1, wn1)


def _dense2(hs, q, degt, b1):
    return pl.pallas_call(
        _dense2_body,
        grid=(_GRID,),
        in_specs=[
            pl.BlockSpec((_BLK, 128), lambda i: (i, 0)),
            pl.BlockSpec((_NC, _BLK, 128), lambda i: (0, i, 0)),
            pl.BlockSpec((_BLK, _NC), lambda i: (i, 0)),
            pl.BlockSpec((1, 128), lambda i: (0, 0)),
        ],
        out_specs=pl.BlockSpec((_BLK, 128), lambda i: (i, 0)),
        out_shape=jax.ShapeDtypeStruct((_N, 128), jnp.float32),
    )(hs, q, degt, b1)


def kernel(feat, edge_index, W_self0, W_neigh0, b0, ln_gamma, ln_beta,
           W_self1, W_neigh1, b1):
    src = edge_index[0].reshape(_NW, _EPW)
    dst = edge_index[1].reshape(_NW, _NCHUNK, _CHUNK)
    p1, deg0, deg1 = _seg_with_deg(feat, src, dst)
    degt = jnp.concatenate([deg0[:, None], deg1[:, None]], axis=1)
    hs, hw = _dense1(feat, p1, degt, W_self0, W_neigh0,
                     b0.reshape(1, -1), ln_gamma.reshape(1, -1),
                     ln_beta.reshape(1, -1), W_self1, W_neigh1)
    (q,) = _seg_no_deg(hw, src, dst)
    return _dense2(hs, q, degt, b1.reshape(1, -1))


# speedup vs baseline: 11.8361x; 1.0130x over previous
import functools

import jax
import jax.numpy as jnp
from jax import lax
from jax.experimental import pallas as pl
from jax.experimental.pallas import tpu as pltpu
from jax.experimental.pallas import tpu_sc as plsc

_N = 10000
_E = 320000
_D = 128
_NC = 2
_NS = 16
_NW = _NC * _NS
_EPW = _E // _NW
_CHUNK = 80
_NCHUNK = _EPW // _CHUNK
_WB = 80
_NWB = _N // _WB


def _seg_body(with_deg, *refs):
    if with_deg:
        (val_hbm, src_hbm, dst_hbm, part_hbm, deg0_hbm, deg1_hbm,
         acc_sh, deg_sh, sidx, didx, rows, dstage, ones,
         gsem, ssem, dsem) = refs
    else:
        (val_hbm, src_hbm, dst_hbm, part_hbm,
         acc_sh, sidx, didx, rows, gsem, ssem) = refs
    c = lax.axis_index("c")
    s = lax.axis_index("s")
    wid = s * _NC + c

    zv = jnp.zeros((16,), jnp.float32)

    pltpu.sync_copy(src_hbm.at[wid], sidx)
    pltpu.sync_copy(dst_hbm.at[wid], didx)

    def _zrow(r, _):
        def _zcol(q, _2):
            rows[0, r, pl.ds(q * 16, 16)] = zv
            return 0
        lax.fori_loop(0, _D // 16, _zcol, 0)
        return 0
    lax.fori_loop(0, _CHUNK, _zrow, 0)

    def _z_desc(idx, b):
        return pltpu.make_async_copy(rows.at[0],
                                     acc_sh.at[pl.ds(idx * _WB, _WB)],
                                     gsem.at[b])
    for k in range(8):
        idx = s + _NS * k
        @pl.when(idx < _NWB)
        def _():
            if k >= 2:
                _z_desc(s + _NS * (k - 2), k % 2).wait()
            _z_desc(idx, k % 2).start()
    for k in (5, 6, 7):
        idx = s + _NS * k
        fired = idx < _NWB
        next2 = s + _NS * (k + 2) < _NWB
        @pl.when(jnp.logical_and(fired, jnp.logical_not(next2)))
        def _():
            _z_desc(idx, k % 2).wait()

    if with_deg:
        def _zdeg(i, _):
            dstage[pl.ds(i * 16, 16)] = zv
            return 0
        lax.fori_loop(0, 63, _zdeg, 0)
        def _fones(i, _):
            ones[pl.ds(i * 16, 16)] = zv + 1.0
            return 0
        lax.fori_loop(0, _CHUNK // 16, _fones, 0)

        @pl.when(s < 10)
        def _():
            pltpu.sync_copy(dstage.at[pl.ds(0, 1000)],
                            deg_sh.at[pl.ds(s * 1000, 1000)])

    plsc.subcore_barrier()

    def _g_desc(j, b):
        return pltpu.make_async_copy(
            val_hbm.at[sidx.at[pl.ds(j * _CHUNK, _CHUNK)]], rows.at[b],
            gsem.at[b])

    def _s_desc(j, b):
        return pltpu.make_async_copy(rows.at[b], acc_sh.at[didx.at[j]],
                                     ssem.at[b])

    def _d_desc(j, b):
        return pltpu.make_async_copy(ones, deg_sh.at[didx.at[j]],
                                     dsem.at[b])

    def _slot(j, b, drain, fire):
        _g_desc(j, b).wait()
        _s_desc(j, b).start(add=True)
        if with_deg:
            _d_desc(j, b).start(add=True)
        if drain:
            _s_desc(j - 1, 1 - b).wait()
            if with_deg:
                _d_desc(j - 1, 1 - b).wait()
        if fire:
            _g_desc(j + 1, 1 - b).start()

    _g_desc(0, 0).start()
    _slot(0, 0, False, True)

    def _mid(t, _):
        _slot(2 * t + 1, 1, True, True)
        _slot(2 * t + 2, 0, True, True)
        return 0
    lax.fori_loop(0, (_NCHUNK - 3) // 2, _mid, 0)

    _slot(_NCHUNK - 2, 1, True, True)
    _slot(_NCHUNK - 1, 0, True, False)
    _s_desc(_NCHUNK - 1, 0).wait()
    if with_deg:
        _d_desc(_NCHUNK - 1, 0).wait()

    plsc.subcore_barrier()

    def _wr_desc(idx, b):
        return pltpu.make_async_copy(acc_sh.at[pl.ds(idx * _WB, _WB)],
                                     rows.at[b], gsem.at[b])

    def _ww_desc(idx, b):
        return pltpu.make_async_copy(rows.at[b],
                                     part_hbm.at[c, pl.ds(idx * _WB, _WB)],
                                     ssem.at[b])
    for k in range(8):
        idx = s + _NS * k
        @pl.when(idx < _NWB)
        def _():
            if k >= 2:
                _ww_desc(s + _NS * (k - 2), k % 2).wait()
            _wr_desc(idx, k % 2).start()
            _wr_desc(idx, k % 2).wait()
            _ww_desc(idx, k % 2).start()
    for k in (5, 6, 7):
        idx = s + _NS * k
        fired = idx < _NWB
        next2 = s + _NS * (k + 2) < _NWB
        @pl.when(jnp.logical_and(fired, jnp.logical_not(next2)))
        def _():
            _ww_desc(idx, k % 2).wait()

    if with_deg:
        @pl.when(s < 10)
        def _():
            pltpu.sync_copy(deg_sh.at[pl.ds(s * 1000, 1000)],
                            dstage.at[pl.ds(0, 1000)])

            @pl.when(c == 0)
            def _():
                pltpu.sync_copy(dstage.at[pl.ds(0, 1000)],
                                deg0_hbm.at[pl.ds(s * 1000, 1000)])

            @pl.when(c == 1)
            def _():
                pltpu.sync_copy(dstage.at[pl.ds(0, 1000)],
                                deg1_hbm.at[pl.ds(s * 1000, 1000)])


def _make_seg_kernel(with_deg):
    out_type = [jax.ShapeDtypeStruct((_NC, _N, _D), jnp.float32)]
    scratch = [
        pltpu.VMEM_SHARED((_N, _D), jnp.float32),
    ]
    if with_deg:
        out_type.append(jax.ShapeDtypeStruct((_N,), jnp.float32))
        out_type.append(jax.ShapeDtypeStruct((_N,), jnp.float32))
        scratch.append(pltpu.VMEM_SHARED((_N,), jnp.float32))
    scratch += [
        pltpu.VMEM((_EPW,), jnp.int32),
        pltpu.VMEM((_NCHUNK, _CHUNK), jnp.int32),
        pltpu.VMEM((2, _CHUNK, _D), jnp.float32),
    ]
    if with_deg:
        scratch += [
            pltpu.VMEM((1008,), jnp.float32),
            pltpu.VMEM((_CHUNK,), jnp.float32),
        ]
    scratch.append(pltpu.SemaphoreType.DMA((2,)))
    scratch.append(pltpu.SemaphoreType.DMA((2,)))
    if with_deg:
        scratch.append(pltpu.SemaphoreType.DMA((2,)))
    mesh = plsc.VectorSubcoreMesh(core_axis_name="c", subcore_axis_name="s",
                                  num_cores=_NC, num_subcores=_NS)
    return pl.kernel(functools.partial(_seg_body, with_deg),
                     out_type=out_type, mesh=mesh, scratch_types=scratch)


_seg_with_deg = _make_seg_kernel(True)
_seg_no_deg = _make_seg_kernel(False)


def _dense1_body(feat_ref, p_ref, degt_ref, ws0_ref, wn0_ref, b0_ref,
                 g_ref, be_ref, ws1_ref, wn1_ref, hs_ref, hw_ref):
    deg = degt_ref[...]
    inv = 1.0 / jnp.maximum(deg[:, 0:1] + deg[:, 1:2], 1.0)
    agg = (p_ref[0] + p_ref[1]) * inv
    z = (jnp.dot(feat_ref[...], ws0_ref[...],
                 preferred_element_type=jnp.float32)
         + jnp.dot(agg, wn0_ref[...], preferred_element_type=jnp.float32)
         + b0_ref[...])
    mu = jnp.mean(z, axis=-1, keepdims=True)
    var = jnp.mean((z - mu) ** 2, axis=-1, keepdims=True)
    h = (z - mu) * lax.rsqrt(var + 1e-5) * g_ref[...] + be_ref[...]
    h = jnp.maximum(h, 0.0)
    hs_ref[...] = jnp.dot(h, ws1_ref[...], preferred_element_type=jnp.float32)
    hw_ref[...] = jnp.dot(h, wn1_ref[...], preferred_element_type=jnp.float32)


def _dense2_body(hs_ref, q_ref, degt_ref, b1_ref, out_ref):
    deg = degt_ref[...]
    inv = 1.0 / jnp.maximum(deg[:, 0:1] + deg[:, 1:2], 1.0)
    out_ref[...] = hs_ref[...] + (q_ref[0] + q_ref[1]) * inv + b1_ref[...]


_BLK = 1000
_GRID = _N // _BLK


def _dense1(feat, p, degt, ws0, wn0, b0, g, be, ws1, wn1):
    return pl.pallas_call(
        _dense1_body,
        grid=(_GRID,),
        in_specs=[
            pl.BlockSpec((_BLK, 128), lambda i: (i, 0)),
            pl.BlockSpec((_NC, _BLK, 128), lambda i: (0, i, 0)),
            pl.BlockSpec((_BLK, _NC), lambda i: (i, 0)),
            pl.BlockSpec((128, 256), lambda i: (0, 0)),
            pl.BlockSpec((128, 256), lambda i: (0, 0)),
            pl.BlockSpec((1, 256), lambda i: (0, 0)),
            pl.BlockSpec((1, 256), lambda i: (0, 0)),
            pl.BlockSpec((1, 256), lambda i: (0, 0)),
            pl.BlockSpec((256, 128), lambda i: (0, 0)),
            pl.BlockSpec((256, 128), lambda i: (0, 0)),
        ],
        out_specs=[
            pl.BlockSpec((_BLK, 128), lambda i: (i, 0)),
            pl.BlockSpec((_BLK, 128), lambda i: (i, 0)),
        ],
        out_shape=[
            jax.ShapeDtypeStruct((_N, 128), jnp.float32),
            jax.ShapeDtypeStruct((_N, 128), jnp.float32),
        ],
    )(feat, p, degt, ws0, wn0, b0, g, be, ws1, wn1)


def _dense2(hs, q, degt, b1):
    return pl.pallas_call(
        _dense2_body,
        grid=(_GRID,),
        in_specs=[
            pl.BlockSpec((_BLK, 128), lambda i: (i, 0)),
            pl.BlockSpec((_NC, _BLK, 128), lambda i: (0, i, 0)),
            pl.BlockSpec((_BLK, _NC), lambda i: (i, 0)),
            pl.BlockSpec((1, 128), lambda i: (0, 0)),
        ],
        out_specs=pl.BlockSpec((_BLK, 128), lambda i: (i, 0)),
        out_shape=jax.ShapeDtypeStruct((_N, 128), jnp.float32),
    )(hs, q, degt, b1)


def kernel(feat, edge_index, W_self0, W_neigh0, b0, ln_gamma, ln_beta,
           W_self1, W_neigh1, b1):
    src = edge_index[0].reshape(_NW, _EPW)
    dst = edge_index[1].reshape(_NW, _NCHUNK, _CHUNK)
    p1, deg0, deg1 = _seg_with_deg(feat, src, dst)
    degt = jnp.concatenate([deg0[:, None], deg1[:, None]], axis=1)
    hs, hw = _dense1(feat, p1, degt, W_self0, W_neigh0,
                     b0.reshape(1, -1), ln_gamma.reshape(1, -1),
                     ln_beta.reshape(1, -1), W_self1, W_neigh1)
    (q,) = _seg_no_deg(hw, src, dst)
    return _dense2(hs, q, degt, b1.reshape(1, -1))


# speedup vs baseline: 11.8701x; 1.0029x over previous
import functools

import jax
import jax.numpy as jnp
from jax import lax
from jax.experimental import pallas as pl
from jax.experimental.pallas import tpu as pltpu
from jax.experimental.pallas import tpu_sc as plsc

_N = 10000
_E = 320000
_D = 128
_NC = 2
_NS = 16
_NW = _NC * _NS
_EPW = _E // _NW
_CHUNK = 80
_NCHUNK = _EPW // _CHUNK
_WB = 80
_NWB = _N // _WB


def _seg_body(with_deg, *refs):
    if with_deg:
        (val_hbm, src_hbm, dst_hbm, part_hbm, deg0_hbm, deg1_hbm,
         acc_sh, deg_sh, sidx, didx, rows, dstage, ones,
         gsem, ssem, dsem) = refs
    else:
        (val_hbm, src_hbm, dst_hbm, part_hbm,
         acc_sh, sidx, didx, rows, gsem, ssem) = refs
    c = lax.axis_index("c")
    s = lax.axis_index("s")
    wid = s * _NC + c

    zv = jnp.zeros((16,), jnp.float32)

    pltpu.sync_copy(src_hbm.at[wid], sidx)
    pltpu.sync_copy(dst_hbm.at[wid], didx)

    def _zrow(r, _):
        def _zcol(q, _2):
            rows[0, r, pl.ds(q * 16, 16)] = zv
            return 0
        lax.fori_loop(0, _D // 16, _zcol, 0)
        return 0
    lax.fori_loop(0, _CHUNK, _zrow, 0)

    def _z_desc(idx, b):
        return pltpu.make_async_copy(rows.at[0],
                                     acc_sh.at[pl.ds(idx * _WB, _WB)],
                                     gsem.at[b])
    for k in range(8):
        idx = s + _NS * k
        @pl.when(idx < _NWB)
        def _():
            if k >= 2:
                _z_desc(s + _NS * (k - 2), k % 2).wait()
            _z_desc(idx, k % 2).start()
    for k in (5, 6, 7):
        idx = s + _NS * k
        fired = idx < _NWB
        next2 = s + _NS * (k + 2) < _NWB
        @pl.when(jnp.logical_and(fired, jnp.logical_not(next2)))
        def _():
            _z_desc(idx, k % 2).wait()

    if with_deg:
        def _zdeg(i, _):
            dstage[pl.ds(i * 16, 16)] = zv
            return 0
        lax.fori_loop(0, 63, _zdeg, 0)
        def _fones(i, _):
            ones[pl.ds(i * 16, 16)] = zv + 1.0
            return 0
        lax.fori_loop(0, _CHUNK // 16, _fones, 0)

        @pl.when(s < 10)
        def _():
            pltpu.sync_copy(dstage.at[pl.ds(0, 1000)],
                            deg_sh.at[pl.ds(s * 1000, 1000)])

    plsc.subcore_barrier()

    def _g_desc(j, b):
        return pltpu.make_async_copy(
            val_hbm.at[sidx.at[pl.ds(j * _CHUNK, _CHUNK)]], rows.at[b],
            gsem.at[b])

    def _s_desc(j, b):
        return pltpu.make_async_copy(rows.at[b], acc_sh.at[didx.at[j]],
                                     ssem.at[b])

    def _d_desc(j, b):
        return pltpu.make_async_copy(ones, deg_sh.at[didx.at[j]],
                                     dsem.at[b])

    def _slot(j, b, drain, fire):
        bn = 1 - b
        _g_desc(j, b).wait()
        _s_desc(j, b).start(add=True)
        if with_deg:
            _d_desc(j, b).start(add=True)
        if drain:
            _s_desc(j - 1, bn).wait()
            if with_deg:
                _d_desc(j - 1, bn).wait()
        if fire:
            _g_desc(j + 1, bn).start()

    _g_desc(0, 0).start()
    _slot(0, 0, False, True)

    def _mid(t, _):
        _slot(2 * t + 1, 1, True, True)
        _slot(2 * t + 2, 0, True, True)
        return 0
    lax.fori_loop(0, (_NCHUNK - 3) // 2, _mid, 0)

    _slot(_NCHUNK - 2, 1, True, True)
    _slot(_NCHUNK - 1, 0, True, False)
    _s_desc(_NCHUNK - 1, 0).wait()
    if with_deg:
        _d_desc(_NCHUNK - 1, 0).wait()

    plsc.subcore_barrier()

    def _wr_desc(idx, b):
        return pltpu.make_async_copy(acc_sh.at[pl.ds(idx * _WB, _WB)],
                                     rows.at[b], gsem.at[b])

    def _ww_desc(idx, b):
        return pltpu.make_async_copy(rows.at[b],
                                     part_hbm.at[c, pl.ds(idx * _WB, _WB)],
                                     ssem.at[b])
    for k in range(8):
        idx = s + _NS * k
        @pl.when(idx < _NWB)
        def _():
            if k >= 2:
                _ww_desc(s + _NS * (k - 2), k % 2).wait()
            _wr_desc(idx, k % 2).start()
            _wr_desc(idx, k % 2).wait()
            _ww_desc(idx, k % 2).start()
    for k in (5, 6, 7):
        idx = s + _NS * k
        fired = idx < _NWB
        next2 = s + _NS * (k + 2) < _NWB
        @pl.when(jnp.logical_and(fired, jnp.logical_not(next2)))
        def _():
            _ww_desc(idx, k % 2).wait()

    if with_deg:
        @pl.when(s < 10)
        def _():
            pltpu.sync_copy(deg_sh.at[pl.ds(s * 1000, 1000)],
                            dstage.at[pl.ds(0, 1000)])

            @pl.when(c == 0)
            def _():
                pltpu.sync_copy(dstage.at[pl.ds(0, 1000)],
                                deg0_hbm.at[pl.ds(s * 1000, 1000)])

            @pl.when(c == 1)
            def _():
                pltpu.sync_copy(dstage.at[pl.ds(0, 1000)],
                                deg1_hbm.at[pl.ds(s * 1000, 1000)])


def _make_seg_kernel(with_deg):
    out_type = [jax.ShapeDtypeStruct((_NC, _N, _D), jnp.float32)]
    scratch = [
        pltpu.VMEM_SHARED((_N, _D), jnp.float32),
    ]
    if with_deg:
        out_type.append(jax.ShapeDtypeStruct((_N,), jnp.float32))
        out_type.append(jax.ShapeDtypeStruct((_N,), jnp.float32))
        scratch.append(pltpu.VMEM_SHARED((_N,), jnp.float32))
    scratch += [
        pltpu.VMEM((_EPW,), jnp.int32),
        pltpu.VMEM((_NCHUNK, _CHUNK), jnp.int32),
        pltpu.VMEM((2, _CHUNK, _D), jnp.float32),
    ]
    if with_deg:
        scratch += [
            pltpu.VMEM((1008,), jnp.float32),
            pltpu.VMEM((_CHUNK,), jnp.float32),
        ]
    scratch.append(pltpu.SemaphoreType.DMA((2,)))
    scratch.append(pltpu.SemaphoreType.DMA((2,)))
    if with_deg:
        scratch.append(pltpu.SemaphoreType.DMA((2,)))
    mesh = plsc.VectorSubcoreMesh(core_axis_name="c", subcore_axis_name="s",
                                  num_cores=_NC, num_subcores=_NS)
    return pl.kernel(functools.partial(_seg_body, with_deg),
                     out_type=out_type, mesh=mesh, scratch_types=scratch)


_seg_with_deg = _make_seg_kernel(True)
_seg_no_deg = _make_seg_kernel(False)


def _dense1_body(feat_ref, p_ref, degt_ref, ws0_ref, wn0_ref, b0_ref,
                 g_ref, be_ref, ws1_ref, wn1_ref, hs_ref, hw_ref):
    deg = degt_ref[...]
    inv = 1.0 / jnp.maximum(deg[:, 0:1] + deg[:, 1:2], 1.0)
    agg = (p_ref[0] + p_ref[1]) * inv
    z = (jnp.dot(feat_ref[...], ws0_ref[...],
                 preferred_element_type=jnp.float32)
         + jnp.dot(agg, wn0_ref[...], preferred_element_type=jnp.float32)
         + b0_ref[...])
    mu = jnp.mean(z, axis=-1, keepdims=True)
    var = jnp.mean((z - mu) ** 2, axis=-1, keepdims=True)
    h = (z - mu) * lax.rsqrt(var + 1e-5) * g_ref[...] + be_ref[...]
    h = jnp.maximum(h, 0.0)
    hs_ref[...] = jnp.dot(h, ws1_ref[...], preferred_element_type=jnp.float32)
    hw_ref[...] = jnp.dot(h, wn1_ref[...], preferred_element_type=jnp.float32)


def _dense2_body(hs_ref, q_ref, degt_ref, b1_ref, out_ref):
    deg = degt_ref[...]
    inv = 1.0 / jnp.maximum(deg[:, 0:1] + deg[:, 1:2], 1.0)
    out_ref[...] = hs_ref[...] + (q_ref[0] + q_ref[1]) * inv + b1_ref[...]


_BLK = 1000
_GRID = _N // _BLK


def _dense1(feat, p, degt, ws0, wn0, b0, g, be, ws1, wn1):
    return pl.pallas_call(
        _dense1_body,
        grid=(_GRID,),
        in_specs=[
            pl.BlockSpec((_BLK, 128), lambda i: (i, 0)),
            pl.BlockSpec((_NC, _BLK, 128), lambda i: (0, i, 0)),
            pl.BlockSpec((_BLK, _NC), lambda i: (i, 0)),
            pl.BlockSpec((128, 256), lambda i: (0, 0)),
            pl.BlockSpec((128, 256), lambda i: (0, 0)),
            pl.BlockSpec((1, 256), lambda i: (0, 0)),
            pl.BlockSpec((1, 256), lambda i: (0, 0)),
            pl.BlockSpec((1, 256), lambda i: (0, 0)),
            pl.BlockSpec((256, 128), lambda i: (0, 0)),
            pl.BlockSpec((256, 128), lambda i: (0, 0)),
        ],
        out_specs=[
            pl.BlockSpec((_BLK, 128), lambda i: (i, 0)),
            pl.BlockSpec((_BLK, 128), lambda i: (i, 0)),
        ],
        out_shape=[
            jax.ShapeDtypeStruct((_N, 128), jnp.float32),
            jax.ShapeDtypeStruct((_N, 128), jnp.float32),
        ],
    )(feat, p, degt, ws0, wn0, b0, g, be, ws1, wn1)


def _dense2(hs, q, degt, b1):
    return pl.pallas_call(
        _dense2_body,
        grid=(_GRID,),
        in_specs=[
            pl.BlockSpec((_BLK, 128), lambda i: (i, 0)),
            pl.BlockSpec((_NC, _BLK, 128), lambda i: (0, i, 0)),
            pl.BlockSpec((_BLK, _NC), lambda i: (i, 0)),
            pl.BlockSpec((1, 128), lambda i: (0, 0)),
        ],
        out_specs=pl.BlockSpec((_BLK, 128), lambda i: (i, 0)),
        out_shape=jax.ShapeDtypeStruct((_N, 128), jnp.float32),
    )(hs, q, degt, b1)


def kernel(feat, edge_index, W_self0, W_neigh0, b0, ln_gamma, ln_beta,
           W_self1, W_neigh1, b1):
    src = edge_index[0].reshape(_NW, _EPW)
    dst = edge_index[1].reshape(_NW, _NCHUNK, _CHUNK)
    p1, deg0, deg1 = _seg_with_deg(feat, src, dst)
    degt = jnp.concatenate([deg0[:, None], deg1[:, None]], axis=1)
    hs, hw = _dense1(feat, p1, degt, W_self0, W_neigh0,
                     b0.reshape(1, -1), ln_gamma.reshape(1, -1),
                     ln_beta.reshape(1, -1), W_self1, W_neigh1)
    (q,) = _seg_no_deg(hw, src, dst)
    return _dense2(hs, q, degt, b1.reshape(1, -1))


# speedup vs baseline: 16.8746x; 1.4216x over previous
import functools

import jax
import jax.numpy as jnp
from jax import lax
from jax.experimental import pallas as pl
from jax.experimental.pallas import tpu as pltpu
from jax.experimental.pallas import tpu_sc as plsc

_N = 10000
_E = 320000
_D = 128
_NC = 2
_NS = 16
_NW = _NC * _NS
_EPW = _E // _NW
_CHUNK = 80
_NCHUNK = _EPW // _CHUNK
_NBUF = 3
_NIDX = 6
_WB = 80
_NWB = _N // _WB
_NWBS = 8


def _seg_body(with_deg, *refs):
    if with_deg:
        (val_hbm, src_hbm, dst_hbm, part_hbm, deg0_hbm, deg1_hbm,
         acc_sh, deg_sh, sidx, didx, rows, dstage, ones,
         gsem, ssem, dsem, isem) = refs
    else:
        (val_hbm, src_hbm, dst_hbm, part_hbm,
         acc_sh, sidx, didx, rows, gsem, ssem, isem) = refs
    c = lax.axis_index("c")
    s = lax.axis_index("s")
    wid = s * _NC + c

    zv = jnp.zeros((16,), jnp.float32)

    pltpu.sync_copy(dst_hbm.at[wid], didx)

    def _i_desc(j, m):
        return pltpu.make_async_copy(
            src_hbm.at[pl.ds(wid * _EPW + j * _CHUNK, _CHUNK)],
            sidx.at[m], isem.at[m])

    def _zrow(r, _):
        def _zcol(q, _2):
            rows[0, r, pl.ds(q * 16, 16)] = zv
            return 0
        lax.fori_loop(0, _D // 16, _zcol, 0)
        return 0
    lax.fori_loop(0, _CHUNK, _zrow, 0)

    def _z_desc(idx, b):
        return pltpu.make_async_copy(rows.at[0],
                                     acc_sh.at[pl.ds(idx * _WB, _WB)],
                                     gsem.at[b])
    for k in range(_NWBS):
        idx = s + _NS * k
        @pl.when(idx < _NWB)
        def _():
            if k >= 2:
                _z_desc(s + _NS * (k - 2), k % 2).wait()
            _z_desc(idx, k % 2).start()
    for k in (_NWBS - 3, _NWBS - 2, _NWBS - 1):
        idx = s + _NS * k
        fired = idx < _NWB
        next2 = s + _NS * (k + 2) < _NWB
        @pl.when(jnp.logical_and(fired, jnp.logical_not(next2)))
        def _():
            _z_desc(idx, k % 2).wait()

    if with_deg:
        def _zdeg(i, _):
            dstage[pl.ds(i * 16, 16)] = zv
            return 0
        lax.fori_loop(0, 63, _zdeg, 0)
        def _fones(i, _):
            ones[pl.ds(i * 16, 16)] = zv + 1.0
            return 0
        lax.fori_loop(0, _CHUNK // 16, _fones, 0)

        @pl.when(s < 10)
        def _():
            pltpu.sync_copy(dstage.at[pl.ds(0, 1000)],
                            deg_sh.at[pl.ds(s * 1000, 1000)])

    plsc.subcore_barrier()

    def _g_desc(j, b, m):
        return pltpu.make_async_copy(
            val_hbm.at[sidx.at[m]], rows.at[b], gsem.at[b])

    def _s_desc(j, b):
        return pltpu.make_async_copy(rows.at[b], acc_sh.at[didx.at[j]],
                                     ssem.at[b])

    def _d_desc(j, b):
        return pltpu.make_async_copy(ones, deg_sh.at[didx.at[j]],
                                     dsem.at[b])

    def _slot(j, k, drain, fire, fire_idx):
        b = k % _NBUF
        bp = (b + _NBUF - 1) % _NBUF
        _g_desc(j, b, k).wait()
        _s_desc(j, b).start(add=True)
        if with_deg:
            _d_desc(j, b).start(add=True)
        if fire_idx:
            _i_desc(j + _NIDX, k).start()
        if drain:
            _s_desc(j - 1, bp).wait()
            if with_deg:
                _d_desc(j - 1, bp).wait()
        if fire:
            kg = (k + _NBUF - 1) % _NIDX
            _i_desc(j + _NBUF - 1, kg).wait()
            _g_desc(j + _NBUF - 1, bp, kg).start()

    for m in range(_NIDX):
        _i_desc(m, m).start()
    for b in range(_NBUF - 1):
        _i_desc(b, b).wait()
        _g_desc(b, b, b).start()
    _slot(0, 0, False, True, True)

    def _mid(t, _):
        for k in range(1, _NIDX + 1):
            _slot(_NIDX * t + k, k % _NIDX, True, True, True)
        return 0
    _NLOOP = 19
    lax.fori_loop(0, _NLOOP, _mid, 0)

    for j in range(_NLOOP * _NIDX + 1, _NCHUNK):
        _slot(j, j % _NIDX, True, j + _NBUF - 1 < _NCHUNK,
              j + _NIDX < _NCHUNK)
    _s_desc(_NCHUNK - 1, (_NCHUNK - 1) % _NBUF).wait()
    if with_deg:
        _d_desc(_NCHUNK - 1, (_NCHUNK - 1) % _NBUF).wait()

    plsc.subcore_barrier()

    def _wr_desc(idx, b):
        return pltpu.make_async_copy(acc_sh.at[pl.ds(idx * _WB, _WB)],
                                     rows.at[b], gsem.at[b])

    def _ww_desc(idx, b):
        return pltpu.make_async_copy(rows.at[b],
                                     part_hbm.at[c, pl.ds(idx * _WB, _WB)],
                                     ssem.at[b])
    for k in range(_NWBS):
        idx = s + _NS * k
        @pl.when(idx < _NWB)
        def _():
            if k >= 2:
                _ww_desc(s + _NS * (k - 2), k % 2).wait()
            _wr_desc(idx, k % 2).start()
            _wr_desc(idx, k % 2).wait()
            _ww_desc(idx, k % 2).start()
    for k in (_NWBS - 3, _NWBS - 2, _NWBS - 1):
        idx = s + _NS * k
        fired = idx < _NWB
        next2 = s + _NS * (k + 2) < _NWB
        @pl.when(jnp.logical_and(fired, jnp.logical_not(next2)))
        def _():
            _ww_desc(idx, k % 2).wait()

    if with_deg:
        @pl.when(s < 10)
        def _():
            pltpu.sync_copy(deg_sh.at[pl.ds(s * 1000, 1000)],
                            dstage.at[pl.ds(0, 1000)])

            @pl.when(c == 0)
            def _():
                pltpu.sync_copy(dstage.at[pl.ds(0, 1000)],
                                deg0_hbm.at[pl.ds(s * 1000, 1000)])

            @pl.when(c == 1)
            def _():
                pltpu.sync_copy(dstage.at[pl.ds(0, 1000)],
                                deg1_hbm.at[pl.ds(s * 1000, 1000)])


def _make_seg_kernel(with_deg):
    out_type = [jax.ShapeDtypeStruct((_NC, _N, _D), jnp.float32)]
    scratch = [
        pltpu.VMEM_SHARED((_N, _D), jnp.float32),
    ]
    if with_deg:
        out_type.append(jax.ShapeDtypeStruct((_N,), jnp.float32))
        out_type.append(jax.ShapeDtypeStruct((_N,), jnp.float32))
        scratch.append(pltpu.VMEM_SHARED((_N,), jnp.float32))
    scratch += [
        pltpu.VMEM((_NIDX, _CHUNK), jnp.int32),
        pltpu.VMEM((_NCHUNK, _CHUNK), jnp.int32),
        pltpu.VMEM((_NBUF, _CHUNK, _D), jnp.float32),
    ]
    if with_deg:
        scratch += [
            pltpu.VMEM((1008,), jnp.float32),
            pltpu.VMEM((_CHUNK,), jnp.float32),
        ]
    scratch.append(pltpu.SemaphoreType.DMA((_NBUF,)))
    scratch.append(pltpu.SemaphoreType.DMA((_NBUF,)))
    if with_deg:
        scratch.append(pltpu.SemaphoreType.DMA((_NBUF,)))
    scratch.append(pltpu.SemaphoreType.DMA((_NIDX,)))
    mesh = plsc.VectorSubcoreMesh(core_axis_name="c", subcore_axis_name="s",
                                  num_cores=_NC, num_subcores=_NS)
    return pl.kernel(functools.partial(_seg_body, with_deg),
                     out_type=out_type, mesh=mesh, scratch_types=scratch)


_seg_with_deg = _make_seg_kernel(True)
_seg_no_deg = _make_seg_kernel(False)


def _dense1_body(feat_ref, p_ref, degt_ref, ws0_ref, wn0_ref, b0_ref,
                 g_ref, be_ref, ws1_ref, wn1_ref, hs_ref, hw_ref):
    deg = degt_ref[...]
    inv = 1.0 / jnp.maximum(deg[:, 0:1] + deg[:, 1:2], 1.0)
    agg = (p_ref[0] + p_ref[1]) * inv
    z = (jnp.dot(feat_ref[...], ws0_ref[...],
                 preferred_element_type=jnp.float32)
         + jnp.dot(agg, wn0_ref[...], preferred_element_type=jnp.float32)
         + b0_ref[...])
    mu = jnp.mean(z, axis=-1, keepdims=True)
    var = jnp.mean((z - mu) ** 2, axis=-1, keepdims=True)
    h = (z - mu) * lax.rsqrt(var + 1e-5) * g_ref[...] + be_ref[...]
    h = jnp.maximum(h, 0.0)
    hs_ref[...] = jnp.dot(h, ws1_ref[...], preferred_element_type=jnp.float32)
    hw_ref[...] = jnp.dot(h, wn1_ref[...], preferred_element_type=jnp.float32)


def _dense2_body(hs_ref, q_ref, degt_ref, b1_ref, out_ref):
    deg = degt_ref[...]
    inv = 1.0 / jnp.maximum(deg[:, 0:1] + deg[:, 1:2], 1.0)
    out_ref[...] = hs_ref[...] + (q_ref[0] + q_ref[1]) * inv + b1_ref[...]


_BLK = 1000
_GRID = _N // _BLK


def _dense1(feat, p, degt, ws0, wn0, b0, g, be, ws1, wn1):
    return pl.pallas_call(
        _dense1_body,
        grid=(_GRID,),
        in_specs=[
            pl.BlockSpec((_BLK, 128), lambda i: (i, 0)),
            pl.BlockSpec((_NC, _BLK, 128), lambda i: (0, i, 0)),
            pl.BlockSpec((_BLK, _NC), lambda i: (i, 0)),
            pl.BlockSpec((128, 256), lambda i: (0, 0)),
            pl.BlockSpec((128, 256), lambda i: (0, 0)),
            pl.BlockSpec((1, 256), lambda i: (0, 0)),
            pl.BlockSpec((1, 256), lambda i: (0, 0)),
            pl.BlockSpec((1, 256), lambda i: (0, 0)),
            pl.BlockSpec((256, 128), lambda i: (0, 0)),
            pl.BlockSpec((256, 128), lambda i: (0, 0)),
        ],
        out_specs=[
            pl.BlockSpec((_BLK, 128), lambda i: (i, 0)),
            pl.BlockSpec((_BLK, 128), lambda i: (i, 0)),
        ],
        out_shape=[
            jax.ShapeDtypeStruct((_N, 128), jnp.float32),
            jax.ShapeDtypeStruct((_N, 128), jnp.float32),
        ],
    )(feat, p, degt, ws0, wn0, b0, g, be, ws1, wn1)


def _dense2(hs, q, degt, b1):
    return pl.pallas_call(
        _dense2_body,
        grid=(_GRID,),
        in_specs=[
            pl.BlockSpec((_BLK, 128), lambda i: (i, 0)),
            pl.BlockSpec((_NC, _BLK, 128), lambda i: (0, i, 0)),
            pl.BlockSpec((_BLK, _NC), lambda i: (i, 0)),
            pl.BlockSpec((1, 128), lambda i: (0, 0)),
        ],
        out_specs=pl.BlockSpec((_BLK, 128), lambda i: (i, 0)),
        out_shape=jax.ShapeDtypeStruct((_N, 128), jnp.float32),
    )(hs, q, degt, b1)


def kernel(feat, edge_index, W_self0, W_neigh0, b0, ln_gamma, ln_beta,
           W_self1, W_neigh1, b1):
    src = edge_index[0]
    dst = edge_index[1].reshape(_NW, _NCHUNK, _CHUNK)
    p1, deg0, deg1 = _seg_with_deg(feat, src, dst)
    degt = jnp.concatenate([deg0[:, None], deg1[:, None]], axis=1)
    hs, hw = _dense1(feat, p1, degt, W_self0, W_neigh0,
                     b0.reshape(1, -1), ln_gamma.reshape(1, -1),
                     ln_beta.reshape(1, -1), W_self1, W_neigh1)
    (q,) = _seg_no_deg(hw, src, dst)
    return _dense2(hs, q, degt, b1.reshape(1, -1))


# speedup vs baseline: 17.1652x; 1.0172x over previous
import functools

import jax
import jax.numpy as jnp
from jax import lax
from jax.experimental import pallas as pl
from jax.experimental.pallas import tpu as pltpu
from jax.experimental.pallas import tpu_sc as plsc

_N = 10000
_E = 320000
_D = 128
_NC = 2
_NS = 16
_NW = _NC * _NS
_EPW = _E // _NW
_CHUNK = 80
_NCHUNK = _EPW // _CHUNK
_NBUF = 4
_NIDX = 8
_WB = 80
_NWB = _N // _WB
_NWBS = 8


def _seg_body(with_deg, *refs):
    if with_deg:
        (val_hbm, src_hbm, dst_hbm, part_hbm, deg0_hbm, deg1_hbm,
         acc_sh, deg_sh, sidx, didx, rows, dstage, ones,
         gsem, ssem, dsem, isem, jsem) = refs
    else:
        (val_hbm, src_hbm, dst_hbm, part_hbm,
         acc_sh, sidx, didx, rows, gsem, ssem, isem, jsem) = refs
    c = lax.axis_index("c")
    s = lax.axis_index("s")
    wid = s * _NC + c

    zv = jnp.zeros((16,), jnp.float32)

    def _i_desc(j, m):
        return pltpu.make_async_copy(
            src_hbm.at[pl.ds(wid * _EPW + j * _CHUNK, _CHUNK)],
            sidx.at[m], isem.at[m])

    def _j_desc(j, m):
        return pltpu.make_async_copy(
            dst_hbm.at[pl.ds(wid * _EPW + j * _CHUNK, _CHUNK)],
            didx.at[m], jsem.at[m])

    for m in range(_NIDX):
        _i_desc(m, m).start()
        _j_desc(m, m).start()

    def _zrow(r, _):
        def _zcol(q, _2):
            rows[0, r, pl.ds(q * 16, 16)] = zv
            return 0
        lax.fori_loop(0, _D // 16, _zcol, 0)
        return 0
    lax.fori_loop(0, _CHUNK, _zrow, 0)

    def _z_desc(idx, b):
        return pltpu.make_async_copy(rows.at[0],
                                     acc_sh.at[pl.ds(idx * _WB, _WB)],
                                     gsem.at[b])
    for k in range(_NWBS):
        idx = s + _NS * k
        @pl.when(idx < _NWB)
        def _():
            if k >= 2:
                _z_desc(s + _NS * (k - 2), k % 2).wait()
            _z_desc(idx, k % 2).start()
    for k in (_NWBS - 3, _NWBS - 2, _NWBS - 1):
        idx = s + _NS * k
        fired = idx < _NWB
        next2 = s + _NS * (k + 2) < _NWB
        @pl.when(jnp.logical_and(fired, jnp.logical_not(next2)))
        def _():
            _z_desc(idx, k % 2).wait()

    if with_deg:
        def _zdeg(i, _):
            dstage[pl.ds(i * 16, 16)] = zv
            return 0
        lax.fori_loop(0, 63, _zdeg, 0)
        def _fones(i, _):
            ones[pl.ds(i * 16, 16)] = zv + 1.0
            return 0
        lax.fori_loop(0, _CHUNK // 16, _fones, 0)

        @pl.when(s < 10)
        def _():
            pltpu.sync_copy(dstage.at[pl.ds(0, 1000)],
                            deg_sh.at[pl.ds(s * 1000, 1000)])

    plsc.subcore_barrier()

    def _g_desc(j, b, m):
        return pltpu.make_async_copy(
            val_hbm.at[sidx.at[m]], rows.at[b], gsem.at[b])

    def _s_desc(j, b, m):
        return pltpu.make_async_copy(rows.at[b], acc_sh.at[didx.at[m]],
                                     ssem.at[b])

    def _d_desc(j, b, m):
        return pltpu.make_async_copy(ones, deg_sh.at[didx.at[m]],
                                     dsem.at[b])

    def _slot(j, k, drain, fire, fire_i, fire_j):
        b = k % _NBUF
        bp = (b + _NBUF - 1) % _NBUF
        kp = (k + _NIDX - 1) % _NIDX
        _g_desc(j, b, k).wait()
        _j_desc(j, k).wait()
        _s_desc(j, b, k).start(add=True)
        if with_deg:
            _d_desc(j, b, k).start(add=True)
        if fire_i:
            _i_desc(j + _NIDX, k).start()
        if drain:
            _s_desc(j - 1, bp, kp).wait()
            if with_deg:
                _d_desc(j - 1, bp, kp).wait()
        if fire_j:
            _j_desc(j + _NIDX - 1, kp).start()
        if fire:
            kg = (k + _NBUF - 1) % _NIDX
            _i_desc(j + _NBUF - 1, kg).wait()
            _g_desc(j + _NBUF - 1, bp, kg).start()

    for b in range(_NBUF - 1):
        _i_desc(b, b).wait()
        _g_desc(b, b, b).start()
    _slot(0, 0, False, True, True, False)

    def _mid(t, _):
        for k in range(1, _NIDX + 1):
            _slot(_NIDX * t + k, k % _NIDX, True, True, True, True)
        return 0
    _NLOOP = 14
    lax.fori_loop(0, _NLOOP, _mid, 0)

    for j in range(_NLOOP * _NIDX + 1, _NCHUNK):
        _slot(j, j % _NIDX, True, j + _NBUF - 1 < _NCHUNK,
              j + _NIDX < _NCHUNK, j + _NIDX - 1 < _NCHUNK)
    _s_desc(_NCHUNK - 1, (_NCHUNK - 1) % _NBUF,
            (_NCHUNK - 1) % _NIDX).wait()
    if with_deg:
        _d_desc(_NCHUNK - 1, (_NCHUNK - 1) % _NBUF,
                (_NCHUNK - 1) % _NIDX).wait()

    plsc.subcore_barrier()

    def _wr_desc(idx, b):
        return pltpu.make_async_copy(acc_sh.at[pl.ds(idx * _WB, _WB)],
                                     rows.at[b], gsem.at[b])

    def _ww_desc(idx, b):
        return pltpu.make_async_copy(rows.at[b],
                                     part_hbm.at[c, pl.ds(idx * _WB, _WB)],
                                     ssem.at[b])
    for k in range(_NWBS):
        idx = s + _NS * k
        @pl.when(idx < _NWB)
        def _():
            if k >= 2:
                _ww_desc(s + _NS * (k - 2), k % 2).wait()
            _wr_desc(idx, k % 2).start()
            _wr_desc(idx, k % 2).wait()
            _ww_desc(idx, k % 2).start()
    for k in (_NWBS - 3, _NWBS - 2, _NWBS - 1):
        idx = s + _NS * k
        fired = idx < _NWB
        next2 = s + _NS * (k + 2) < _NWB
        @pl.when(jnp.logical_and(fired, jnp.logical_not(next2)))
        def _():
            _ww_desc(idx, k % 2).wait()

    if with_deg:
        @pl.when(s < 10)
        def _():
            pltpu.sync_copy(deg_sh.at[pl.ds(s * 1000, 1000)],
                            dstage.at[pl.ds(0, 1000)])

            @pl.when(c == 0)
            def _():
                pltpu.sync_copy(dstage.at[pl.ds(0, 1000)],
                                deg0_hbm.at[pl.ds(s * 1000, 1000)])

            @pl.when(c == 1)
            def _():
                pltpu.sync_copy(dstage.at[pl.ds(0, 1000)],
                                deg1_hbm.at[pl.ds(s * 1000, 1000)])


def _make_seg_kernel(with_deg):
    out_type = [jax.ShapeDtypeStruct((_NC, _N, _D), jnp.float32)]
    scratch = [
        pltpu.VMEM_SHARED((_N, _D), jnp.float32),
    ]
    if with_deg:
        out_type.append(jax.ShapeDtypeStruct((_N,), jnp.float32))
        out_type.append(jax.ShapeDtypeStruct((_N,), jnp.float32))
        scratch.append(pltpu.VMEM_SHARED((_N,), jnp.float32))
    scratch += [
        pltpu.VMEM((_NIDX, _CHUNK), jnp.int32),
        pltpu.VMEM((_NIDX, _CHUNK), jnp.int32),
        pltpu.VMEM((_NBUF, _CHUNK, _D), jnp.float32),
    ]
    if with_deg:
        scratch += [
            pltpu.VMEM((1008,), jnp.float32),
            pltpu.VMEM((_CHUNK,), jnp.float32),
        ]
    scratch.append(pltpu.SemaphoreType.DMA((_NBUF,)))
    scratch.append(pltpu.SemaphoreType.DMA((_NBUF,)))
    if with_deg:
        scratch.append(pltpu.SemaphoreType.DMA((_NBUF,)))
    scratch.append(pltpu.SemaphoreType.DMA((_NIDX,)))
    scratch.append(pltpu.SemaphoreType.DMA((_NIDX,)))
    mesh = plsc.VectorSubcoreMesh(core_axis_name="c", subcore_axis_name="s",
                                  num_cores=_NC, num_subcores=_NS)
    return pl.kernel(functools.partial(_seg_body, with_deg),
                     out_type=out_type, mesh=mesh, scratch_types=scratch)


_seg_with_deg = _make_seg_kernel(True)
_seg_no_deg = _make_seg_kernel(False)


def _dense1_body(feat_ref, p_ref, degt_ref, ws0_ref, wn0_ref, b0_ref,
                 g_ref, be_ref, ws1_ref, wn1_ref, hs_ref, hw_ref):
    deg = degt_ref[...]
    inv = 1.0 / jnp.maximum(deg[:, 0:1] + deg[:, 1:2], 1.0)
    agg = (p_ref[0] + p_ref[1]) * inv
    z = (jnp.dot(feat_ref[...], ws0_ref[...],
                 preferred_element_type=jnp.float32)
         + jnp.dot(agg, wn0_ref[...], preferred_element_type=jnp.float32)
         + b0_ref[...])
    mu = jnp.mean(z, axis=-1, keepdims=True)
    var = jnp.mean((z - mu) ** 2, axis=-1, keepdims=True)
    h = (z - mu) * lax.rsqrt(var + 1e-5) * g_ref[...] + be_ref[...]
    h = jnp.maximum(h, 0.0)
    hs_ref[...] = jnp.dot(h, ws1_ref[...], preferred_element_type=jnp.float32)
    hw_ref[...] = jnp.dot(h, wn1_ref[...], preferred_element_type=jnp.float32)


def _dense2_body(hs_ref, q_ref, degt_ref, b1_ref, out_ref):
    deg = degt_ref[...]
    inv = 1.0 / jnp.maximum(deg[:, 0:1] + deg[:, 1:2], 1.0)
    out_ref[...] = hs_ref[...] + (q_ref[0] + q_ref[1]) * inv + b1_ref[...]


_BLK = 1000
_GRID = _N // _BLK


def _dense1(feat, p, degt, ws0, wn0, b0, g, be, ws1, wn1):
    return pl.pallas_call(
        _dense1_body,
        grid=(_GRID,),
        in_specs=[
            pl.BlockSpec((_BLK, 128), lambda i: (i, 0)),
            pl.BlockSpec((_NC, _BLK, 128), lambda i: (0, i, 0)),
            pl.BlockSpec((_BLK, _NC), lambda i: (i, 0)),
            pl.BlockSpec((128, 256), lambda i: (0, 0)),
            pl.BlockSpec((128, 256), lambda i: (0, 0)),
            pl.BlockSpec((1, 256), lambda i: (0, 0)),
            pl.BlockSpec((1, 256), lambda i: (0, 0)),
            pl.BlockSpec((1, 256), lambda i: (0, 0)),
            pl.BlockSpec((256, 128), lambda i: (0, 0)),
            pl.BlockSpec((256, 128), lambda i: (0, 0)),
        ],
        out_specs=[
            pl.BlockSpec((_BLK, 128), lambda i: (i, 0)),
            pl.BlockSpec((_BLK, 128), lambda i: (i, 0)),
        ],
        out_shape=[
            jax.ShapeDtypeStruct((_N, 128), jnp.float32),
            jax.ShapeDtypeStruct((_N, 128), jnp.float32),
        ],
    )(feat, p, degt, ws0, wn0, b0, g, be, ws1, wn1)


def _dense2(hs, q, degt, b1):
    return pl.pallas_call(
        _dense2_body,
        grid=(_GRID,),
        in_specs=[
            pl.BlockSpec((_BLK, 128), lambda i: (i, 0)),
            pl.BlockSpec((_NC, _BLK, 128), lambda i: (0, i, 0)),
            pl.BlockSpec((_BLK, _NC), lambda i: (i, 0)),
            pl.BlockSpec((1, 128), lambda i: (0, 0)),
        ],
        out_specs=pl.BlockSpec((_BLK, 128), lambda i: (i, 0)),
        out_shape=jax.ShapeDtypeStruct((_N, 128), jnp.float32),
    )(hs, q, degt, b1)


def kernel(feat, edge_index, W_self0, W_neigh0, b0, ln_gamma, ln_beta,
           W_self1, W_neigh1, b1):
    src = edge_index[0]
    dst = edge_index[1]
    p1, deg0, deg1 = _seg_with_deg(feat, src, dst)
    degt = jnp.concatenate([deg0[:, None], deg1[:, None]], axis=1)
    hs, hw = _dense1(feat, p1, degt, W_self0, W_neigh0,
                     b0.reshape(1, -1), ln_gamma.reshape(1, -1),
                     ln_beta.reshape(1, -1), W_self1, W_neigh1)
    (q,) = _seg_no_deg(hw, src, dst)
    return _dense2(hs, q, degt, b1.reshape(1, -1))
